# Initial kernel scaffold; baseline (speedup 1.0000x reference)
import functools

import jax
import jax.numpy as jnp
from jax import lax
from jax.experimental import pallas as pl
from jax.experimental.pallas import tpu as pltpu
from jax.experimental.pallas import tpu_sc as plsc

S = 16
N = 65536
K = 64
NC = 2
NS = 16
L = 16
NW = NC * NS
PW = N // NW
NG = PW // L
R = S * 3
ACC_W = 3328
COLS = ACC_W // NS
F32 = jnp.float32
I32 = jnp.int32

_mesh = plsc.VectorSubcoreMesh(core_axis_name="c", subcore_axis_name="s")


def _rsqrt_nr(x):
    i = lax.bitcast_convert_type(x, I32)
    i = 0x5F3759DF - lax.shift_right_logical(i, 1)
    y = lax.bitcast_convert_type(i, F32)
    for _ in range(3):
        y = y * (1.5 - 0.5 * x * y * y)
    return y


@functools.partial(
    pl.kernel,
    out_type=jax.ShapeDtypeStruct((NC * ACC_W,), F32),
    mesh=_mesh,
    scratch_types=[
        pltpu.VMEM((ACC_W,), F32),
        pltpu.VMEM((PW * 3,), F32),
        pltpu.VMEM((PW,), I32),
        pltpu.VMEM((NS * COLS,), F32),
        pltpu.VMEM((COLS,), F32),
        pltpu.VMEM_SHARED((NS, ACC_W), F32),
    ],
)
def _k1(flow_hbm, imap_hbm, out_hbm, acc, fbuf, idxv, red, colbuf, shared):
    cid = lax.axis_index("c")
    sid = lax.axis_index("s")
    wid = cid * NS + sid
    base = wid * PW
    iota = lax.iota(I32, L)

    def zero_body(j, carry):
        acc[pl.ds(j * L, L)] = jnp.zeros((L,), F32)
        return carry

    lax.fori_loop(0, ACC_W // L, zero_body, 0)

    pltpu.sync_copy(imap_hbm.at[pl.ds(base, PW)], idxv)
    ones = jnp.ones((L,), F32)
    for s in range(S):
        pltpu.sync_copy(flow_hbm.at[s, pl.ds(base * 3, PW * 3)], fbuf)

        def body(g, carry, s=s):
            idx = idxv[pl.ds(g * L, L)]
            if s == 0:
                plsc.addupdate_scatter(acc, [idx + (R * K)], ones)
            for c in range(3):
                f = plsc.load_gather(fbuf, [g * (3 * L) + iota * 3 + c])
                plsc.addupdate_scatter(acc, [idx + ((s * 3 + c) * K)], f)
            return carry

        lax.fori_loop(0, NG, body, 0)

    pltpu.sync_copy(acc, shared.at[sid])
    plsc.subcore_barrier()
    col0 = sid * COLS
    for l in range(NS):
        pltpu.sync_copy(shared.at[l, pl.ds(col0, COLS)],
                        red.at[pl.ds(l * COLS, COLS)])

    def red_body(t, carry):
        v = red[pl.ds(t * L, L)]
        for l in range(1, NS):
            v = v + red[pl.ds(l * COLS + t * L, L)]
        colbuf[pl.ds(t * L, L)] = v
        return carry

    lax.fori_loop(0, COLS // L, red_body, 0)
    pltpu.sync_copy(colbuf, out_hbm.at[pl.ds(cid * ACC_W + col0, COLS)])


@functools.partial(
    pl.kernel,
    out_type=jax.ShapeDtypeStruct((NC * K,), F32),
    mesh=_mesh,
    scratch_types=[
        pltpu.VMEM((NC * ACC_W,), F32),
        pltpu.VMEM((R * K,), F32),
        pltpu.VMEM((K,), F32),
        pltpu.VMEM((PW * 3,), F32),
        pltpu.VMEM((PW,), I32),
        pltpu.VMEM((K,), F32),
        pltpu.VMEM((NS * L,), F32),
        pltpu.VMEM_SHARED((NS, K), F32),
    ],
)
def _k2(flow_hbm, imap_hbm, part_hbm, out_hbm,
        pbuf, means, invc, fbuf, idxv, nacc, red, sharedn):
    cid = lax.axis_index("c")
    sid = lax.axis_index("s")
    wid = cid * NS + sid
    base = wid * PW
    iota = lax.iota(I32, L)

    pltpu.sync_copy(part_hbm, pbuf)
    for q in range(K // L):
        cnt = (pbuf[pl.ds(R * K + q * L, L)]
               + pbuf[pl.ds(ACC_W + R * K + q * L, L)])
        invc[pl.ds(q * L, L)] = 1.0 / jnp.maximum(cnt, 1.0)

    def mean_body(j, carry):
        t = pbuf[pl.ds(j * L, L)] + pbuf[pl.ds(ACC_W + j * L, L)]
        q = lax.rem(j, K // L)
        means[pl.ds(j * L, L)] = t * invc[pl.ds(q * L, L)]
        return carry

    lax.fori_loop(0, R * K // L, mean_body, 0)

    for q in range(K // L):
        nacc[pl.ds(q * L, L)] = jnp.zeros((L,), F32)

    pltpu.sync_copy(imap_hbm.at[pl.ds(base, PW)], idxv)
    for s in range(S):
        pltpu.sync_copy(flow_hbm.at[s, pl.ds(base * 3, PW * 3)], fbuf)

        def body(g, carry, s=s):
            idx = idxv[pl.ds(g * L, L)]
            sq = jnp.zeros((L,), F32)
            for c in range(3):
                f = plsc.load_gather(fbuf, [g * (3 * L) + iota * 3 + c])
                m = plsc.load_gather(means, [idx + ((s * 3 + c) * K)])
                d = m - f
                sq = sq + d * d
            xc = jnp.maximum(sq, 1e-20)
            nrm = sq * _rsqrt_nr(xc)
            plsc.addupdate_scatter(nacc, [idx], nrm)
            return carry

        lax.fori_loop(0, NG, body, 0)

    pltpu.sync_copy(nacc, sharedn.at[sid])
    plsc.subcore_barrier()

    @pl.when(sid < K // L)
    def _():
        col0 = sid * L
        for l in range(NS):
            pltpu.sync_copy(sharedn.at[l, pl.ds(col0, L)],
                            red.at[pl.ds(l * L, L)])
        v = red[pl.ds(0, L)]
        for l in range(1, NS):
            v = v + red[pl.ds(l * L, L)]
        nacc[pl.ds(0, L)] = v
        pltpu.sync_copy(nacc.at[pl.ds(0, L)],
                        out_hbm.at[pl.ds(cid * K + col0, L)])


def _final_body(part_ref, n_ref, out_ref):
    part = part_ref[...]
    nsum = n_ref[...]
    counts = part[0:1, R * K:R * K + K] + part[1:2, R * K:R * K + K]
    ns = nsum[0:1, :] + nsum[1:2, :]
    safe = jnp.maximum(counts, 1.0)
    per = ns / (safe * float(S))
    present = counts > 0.5
    zero = jnp.zeros_like(per)
    tot = jnp.sum(jnp.where(present, per, zero))
    npres = jnp.sum(jnp.where(present, jnp.ones_like(per), zero))
    out_ref[...] = jnp.full((1, 1), tot / (npres - 1.0), F32)


@jax.jit
def kernel(pred_flow, instance_map):
    flow = jnp.reshape(pred_flow, (S, N * 3))
    part = _k1(flow, instance_map)
    nsum = _k2(flow, instance_map, part)
    out = pl.pallas_call(
        _final_body,
        out_shape=jax.ShapeDtypeStruct((1, 1), F32),
    )(jnp.reshape(part, (NC, ACC_W)), jnp.reshape(nsum, (NC, K)))
    return out[0, 0]


# speedup vs baseline: 14.1289x; 14.1289x over previous
import functools

import jax
import jax.numpy as jnp
from jax import lax
from jax.experimental import pallas as pl
from jax.experimental.pallas import tpu as pltpu
from jax.experimental.pallas import tpu_sc as plsc

S = 16
N = 65536
K = 64
NC = 2
NS = 16
L = 16
NW = NC * NS
PW = N // NW
NG = PW // L
R = S * 3
ACC_W = 3328
COLS = ACC_W // NS
F32 = jnp.float32
I32 = jnp.int32

_mesh = plsc.VectorSubcoreMesh(core_axis_name="c", subcore_axis_name="s")


def _rsqrt_nr(x):
    i = lax.bitcast_convert_type(x, I32)
    i = 0x5F3759DF - lax.shift_right_logical(i, 1)
    y = lax.bitcast_convert_type(i, F32)
    for _ in range(3):
        y = y * (1.5 - 0.5 * x * y * y)
    return y


@functools.partial(
    pl.kernel,
    out_type=jax.ShapeDtypeStruct((NC * ACC_W,), F32),
    mesh=_mesh,
    scratch_types=[
        pltpu.VMEM((ACC_W,), F32),
        pltpu.VMEM((PW * 3,), F32),
        pltpu.VMEM((PW,), I32),
        pltpu.VMEM((NS * COLS,), F32),
        pltpu.VMEM((COLS,), F32),
        pltpu.VMEM_SHARED((NS * ACC_W,), F32),
    ],
    compiler_params=pltpu.CompilerParams(needs_layout_passes=False),
)
def _k1(flow_hbm, imap_hbm, out_hbm, acc, fbuf, idxv, red, colbuf, shared):
    cid = lax.axis_index("c")
    sid = lax.axis_index("s")
    wid = cid * NS + sid
    base = wid * PW
    iota = lax.iota(I32, L)

    def zero_body(j, carry):
        acc[pl.ds(j * L, L)] = jnp.zeros((L,), F32)
        return carry

    lax.fori_loop(0, ACC_W // L, zero_body, 0)

    pltpu.sync_copy(imap_hbm.at[pl.ds(base, PW)], idxv)
    ones = jnp.ones((L,), F32)
    for s in range(S):
        pltpu.sync_copy(flow_hbm.at[s, pl.ds(base * 3, PW * 3)], fbuf)

        def body(g, carry, s=s):
            idx = idxv[pl.ds(g * L, L)]
            if s == 0:
                plsc.addupdate_scatter(acc, [idx + (R * K)], ones)
            for c in range(3):
                f = plsc.load_gather(fbuf, [g * (3 * L) + iota * 3 + c])
                plsc.addupdate_scatter(acc, [idx + ((s * 3 + c) * K)], f)
            return carry

        lax.fori_loop(0, NG, body, 0)

    pltpu.sync_copy(acc, shared.at[pl.ds(sid * ACC_W, ACC_W)])
    plsc.subcore_barrier()
    col0 = sid * COLS
    for l in range(NS):
        pltpu.sync_copy(shared.at[pl.ds(l * ACC_W + col0, COLS)],
                        red.at[pl.ds(l * COLS, COLS)])

    def red_body(t, carry):
        v = red[pl.ds(t * L, L)]
        for l in range(1, NS):
            v = v + red[pl.ds(l * COLS + t * L, L)]
        colbuf[pl.ds(t * L, L)] = v
        return carry

    lax.fori_loop(0, COLS // L, red_body, 0)
    pltpu.sync_copy(colbuf, out_hbm.at[pl.ds(cid * ACC_W + col0, COLS)])


@functools.partial(
    pl.kernel,
    out_type=jax.ShapeDtypeStruct((NC * K,), F32),
    mesh=_mesh,
    scratch_types=[
        pltpu.VMEM((NC * ACC_W,), F32),
        pltpu.VMEM((R * K,), F32),
        pltpu.VMEM((K,), F32),
        pltpu.VMEM((PW * 3,), F32),
        pltpu.VMEM((PW,), I32),
        pltpu.VMEM((K,), F32),
        pltpu.VMEM((NS * L,), F32),
        pltpu.VMEM_SHARED((NS * K,), F32),
    ],
    compiler_params=pltpu.CompilerParams(needs_layout_passes=False),
)
def _k2(flow_hbm, imap_hbm, part_hbm, out_hbm,
        pbuf, means, invc, fbuf, idxv, nacc, red, sharedn):
    cid = lax.axis_index("c")
    sid = lax.axis_index("s")
    wid = cid * NS + sid
    base = wid * PW
    iota = lax.iota(I32, L)

    pltpu.sync_copy(part_hbm, pbuf)
    for q in range(K // L):
        cnt = (pbuf[pl.ds(R * K + q * L, L)]
               + pbuf[pl.ds(ACC_W + R * K + q * L, L)])
        invc[pl.ds(q * L, L)] = 1.0 / jnp.maximum(cnt, 1.0)

    def mean_body(j, carry):
        t = pbuf[pl.ds(j * L, L)] + pbuf[pl.ds(ACC_W + j * L, L)]
        q = lax.rem(j, K // L)
        means[pl.ds(j * L, L)] = t * invc[pl.ds(q * L, L)]
        return carry

    lax.fori_loop(0, R * K // L, mean_body, 0)

    for q in range(K // L):
        nacc[pl.ds(q * L, L)] = jnp.zeros((L,), F32)

    pltpu.sync_copy(imap_hbm.at[pl.ds(base, PW)], idxv)
    for s in range(S):
        pltpu.sync_copy(flow_hbm.at[s, pl.ds(base * 3, PW * 3)], fbuf)

        def body(g, carry, s=s):
            idx = idxv[pl.ds(g * L, L)]
            sq = jnp.zeros((L,), F32)
            for c in range(3):
                f = plsc.load_gather(fbuf, [g * (3 * L) + iota * 3 + c])
                m = plsc.load_gather(means, [idx + ((s * 3 + c) * K)])
                d = m - f
                sq = sq + d * d
            xc = jnp.maximum(sq, 1e-20)
            nrm = sq * _rsqrt_nr(xc)
            plsc.addupdate_scatter(nacc, [idx], nrm)
            return carry

        lax.fori_loop(0, NG, body, 0)

    pltpu.sync_copy(nacc, sharedn.at[pl.ds(sid * K, K)])
    plsc.subcore_barrier()

    @pl.when(sid < K // L)
    def _():
        col0 = sid * L
        for l in range(NS):
            pltpu.sync_copy(sharedn.at[pl.ds(l * K + col0, L)],
                            red.at[pl.ds(l * L, L)])
        v = red[pl.ds(0, L)]
        for l in range(1, NS):
            v = v + red[pl.ds(l * L, L)]
        nacc[pl.ds(0, L)] = v
        pltpu.sync_copy(nacc.at[pl.ds(0, L)],
                        out_hbm.at[pl.ds(cid * K + col0, L)])


def _final_body(part_ref, n_ref, out_ref):
    part = part_ref[...]
    nsum = n_ref[...]
    counts = part[0:1, R * K:R * K + K] + part[1:2, R * K:R * K + K]
    ns = nsum[0:1, :] + nsum[1:2, :]
    safe = jnp.maximum(counts, 1.0)
    per = ns / (safe * float(S))
    present = counts > 0.5
    zero = jnp.zeros_like(per)
    tot = jnp.sum(jnp.where(present, per, zero))
    npres = jnp.sum(jnp.where(present, jnp.ones_like(per), zero))
    out_ref[...] = jnp.full((1, 1), tot / (npres - 1.0), F32)


@jax.jit
def kernel(pred_flow, instance_map):
    flow = jnp.reshape(pred_flow, (S, N * 3))
    part = _k1(flow, instance_map)
    nsum = _k2(flow, instance_map, part)
    out = pl.pallas_call(
        _final_body,
        out_shape=jax.ShapeDtypeStruct((1, 1), F32),
    )(jnp.reshape(part, (NC, ACC_W)), jnp.reshape(nsum, (NC, K)))
    return out[0, 0]


# speedup vs baseline: 22.1213x; 1.5657x over previous
import functools

import jax
import jax.numpy as jnp
from jax import lax
from jax.experimental import pallas as pl
from jax.experimental.pallas import tpu as pltpu
from jax.experimental.pallas import tpu_sc as plsc

S = 16
N = 65536
K = 64
NC = 2
NS = 16
L = 16
NW = NC * NS
PW = N // NW
NG = PW // L
R = S * 3
ACC_W = 3328
COLS = ACC_W // NS
F32 = jnp.float32
I32 = jnp.int32

_mesh = plsc.VectorSubcoreMesh(core_axis_name="c", subcore_axis_name="s")


def _rsqrt_nr(x):
    i = lax.bitcast_convert_type(x, I32)
    i = 0x5F3759DF - lax.shift_right_logical(i, 1)
    y = lax.bitcast_convert_type(i, F32)
    for _ in range(3):
        y = y * (1.5 - 0.5 * x * y * y)
    return y


@functools.partial(
    pl.kernel,
    out_type=jax.ShapeDtypeStruct((NC * ACC_W,), F32),
    mesh=_mesh,
    scratch_types=[
        pltpu.VMEM((ACC_W,), F32),
        pltpu.VMEM((3 * PW,), F32),
        pltpu.VMEM((PW,), I32),
        pltpu.VMEM((NS * COLS,), F32),
        pltpu.VMEM((COLS,), F32),
        pltpu.VMEM_SHARED((NS * ACC_W,), F32),
    ],
    compiler_params=pltpu.CompilerParams(
        needs_layout_passes=False, use_tc_tiling_on_sc=False),
)
def _k1(flow_hbm, imap_hbm, out_hbm, acc, fbuf, idxv, red, colbuf, shared):
    cid = lax.axis_index("c")
    sid = lax.axis_index("s")
    wid = cid * NS + sid
    base = wid * PW
    iota = lax.iota(I32, L)

    def zero_body(j, carry):
        acc[pl.ds(j * L, L)] = jnp.zeros((L,), F32)
        return carry

    lax.fori_loop(0, ACC_W // L, zero_body, 0)

    pltpu.sync_copy(imap_hbm.at[pl.ds(base, PW)], idxv)
    ones = jnp.ones((L,), F32)
    for s in range(S):
        for c in range(3):
            pltpu.sync_copy(flow_hbm.at[s, c, pl.ds(base, PW)],
                            fbuf.at[pl.ds(c * PW, PW)])

        def body(g, carry, s=s):
            idx = idxv[pl.ds(g * L, L)]
            if s == 0:
                plsc.addupdate_scatter(acc, [idx + (R * K)], ones)
            for c in range(3):
                f = fbuf[pl.ds(c * PW + g * L, L)]
                plsc.addupdate_scatter(acc, [idx + ((s * 3 + c) * K)], f)
            return carry

        lax.fori_loop(0, NG, body, 0)

    pltpu.sync_copy(acc, shared.at[pl.ds(sid * ACC_W, ACC_W)])
    plsc.subcore_barrier()
    col0 = sid * COLS
    for l in range(NS):
        pltpu.sync_copy(shared.at[pl.ds(l * ACC_W + col0, COLS)],
                        red.at[pl.ds(l * COLS, COLS)])

    def red_body(t, carry):
        v = red[pl.ds(t * L, L)]
        for l in range(1, NS):
            v = v + red[pl.ds(l * COLS + t * L, L)]
        colbuf[pl.ds(t * L, L)] = v
        return carry

    lax.fori_loop(0, COLS // L, red_body, 0)
    pltpu.sync_copy(colbuf, out_hbm.at[pl.ds(cid * ACC_W + col0, COLS)])


@functools.partial(
    pl.kernel,
    out_type=jax.ShapeDtypeStruct((NC * K,), F32),
    mesh=_mesh,
    scratch_types=[
        pltpu.VMEM((NC * ACC_W,), F32),
        pltpu.VMEM((R * K,), F32),
        pltpu.VMEM((K,), F32),
        pltpu.VMEM((3 * PW,), F32),
        pltpu.VMEM((PW,), I32),
        pltpu.VMEM((K,), F32),
        pltpu.VMEM((NS * L,), F32),
        pltpu.VMEM_SHARED((NS * K,), F32),
    ],
    compiler_params=pltpu.CompilerParams(
        needs_layout_passes=False, use_tc_tiling_on_sc=False),
)
def _k2(flow_hbm, imap_hbm, part_hbm, out_hbm,
        pbuf, means, invc, fbuf, idxv, nacc, red, sharedn):
    cid = lax.axis_index("c")
    sid = lax.axis_index("s")
    wid = cid * NS + sid
    base = wid * PW
    iota = lax.iota(I32, L)

    pltpu.sync_copy(part_hbm, pbuf)
    for q in range(K // L):
        cnt = (pbuf[pl.ds(R * K + q * L, L)]
               + pbuf[pl.ds(ACC_W + R * K + q * L, L)])
        invc[pl.ds(q * L, L)] = 1.0 / jnp.maximum(cnt, 1.0)

    def mean_body(j, carry):
        t = pbuf[pl.ds(j * L, L)] + pbuf[pl.ds(ACC_W + j * L, L)]
        q = lax.rem(j, K // L)
        means[pl.ds(j * L, L)] = t * invc[pl.ds(q * L, L)]
        return carry

    lax.fori_loop(0, R * K // L, mean_body, 0)

    for q in range(K // L):
        nacc[pl.ds(q * L, L)] = jnp.zeros((L,), F32)

    pltpu.sync_copy(imap_hbm.at[pl.ds(base, PW)], idxv)
    for s in range(S):
        for c in range(3):
            pltpu.sync_copy(flow_hbm.at[s, c, pl.ds(base, PW)],
                            fbuf.at[pl.ds(c * PW, PW)])

        def body(g, carry, s=s):
            idx = idxv[pl.ds(g * L, L)]
            sq = jnp.zeros((L,), F32)
            for c in range(3):
                f = fbuf[pl.ds(c * PW + g * L, L)]
                m = plsc.load_gather(means, [idx + ((s * 3 + c) * K)])
                d = m - f
                sq = sq + d * d
            xc = jnp.maximum(sq, 1e-20)
            nrm = sq * _rsqrt_nr(xc)
            plsc.addupdate_scatter(nacc, [idx], nrm)
            return carry

        lax.fori_loop(0, NG, body, 0)

    pltpu.sync_copy(nacc, sharedn.at[pl.ds(sid * K, K)])
    plsc.subcore_barrier()

    @pl.when(sid < K // L)
    def _():
        col0 = sid * L
        for l in range(NS):
            pltpu.sync_copy(sharedn.at[pl.ds(l * K + col0, L)],
                            red.at[pl.ds(l * L, L)])
        v = red[pl.ds(0, L)]
        for l in range(1, NS):
            v = v + red[pl.ds(l * L, L)]
        nacc[pl.ds(0, L)] = v
        pltpu.sync_copy(nacc.at[pl.ds(0, L)],
                        out_hbm.at[pl.ds(cid * K + col0, L)])


def _final_body(part_ref, n_ref, out_ref):
    part = part_ref[...]
    nsum = n_ref[...]
    counts = part[0:1, R * K:R * K + K] + part[1:2, R * K:R * K + K]
    ns = nsum[0:1, :] + nsum[1:2, :]
    safe = jnp.maximum(counts, 1.0)
    per = ns / (safe * float(S))
    present = counts > 0.5
    zero = jnp.zeros_like(per)
    tot = jnp.sum(jnp.where(present, per, zero))
    npres = jnp.sum(jnp.where(present, jnp.ones_like(per), zero))
    out_ref[...] = jnp.full((1, 1), tot / (npres - 1.0), F32)


@jax.jit
def kernel(pred_flow, instance_map):
    flow_t = jnp.transpose(pred_flow, (0, 2, 1))
    part = _k1(flow_t, instance_map)
    nsum = _k2(flow_t, instance_map, part)
    out = pl.pallas_call(
        _final_body,
        out_shape=jax.ShapeDtypeStruct((1, 1), F32),
    )(jnp.reshape(part, (NC, ACC_W)), jnp.reshape(nsum, (NC, K)))
    return out[0, 0]


# speedup vs baseline: 29.3039x; 1.3247x over previous
import functools

import jax
import jax.numpy as jnp
from jax import lax
from jax.experimental import pallas as pl
from jax.experimental.pallas import tpu as pltpu
from jax.experimental.pallas import tpu_sc as plsc

S = 16
N = 65536
K = 64
NC = 2
NS = 16
L = 16
NW = NC * NS
PW = N // NW
NG = PW // L
R = S * 3
ACC_W = 3328
COLS = ACC_W // NS
F32 = jnp.float32
I32 = jnp.int32

_mesh = plsc.VectorSubcoreMesh(core_axis_name="c", subcore_axis_name="s")


def _rsqrt_nr(x):
    i = lax.bitcast_convert_type(x, I32)
    i = 0x5F3759DF - lax.shift_right_logical(i, 1)
    y = lax.bitcast_convert_type(i, F32)
    for _ in range(3):
        y = y * (1.5 - 0.5 * x * y * y)
    return y


@functools.partial(
    pl.kernel,
    out_type=jax.ShapeDtypeStruct((NC * ACC_W,), F32),
    mesh=_mesh,
    scratch_types=[
        pltpu.VMEM((ACC_W,), F32),
        pltpu.VMEM((3 * PW,), F32),
        pltpu.VMEM((3 * PW,), F32),
        pltpu.VMEM((PW,), I32),
        pltpu.VMEM((NS * COLS,), F32),
        pltpu.VMEM((COLS,), F32),
        pltpu.VMEM_SHARED((NS * ACC_W,), F32),
        pltpu.SemaphoreType.DMA,
        pltpu.SemaphoreType.DMA,
    ],
    compiler_params=pltpu.CompilerParams(
        needs_layout_passes=False, use_tc_tiling_on_sc=False),
)
def _k1(flow_hbm, imap_hbm, out_hbm, acc, fbuf_a, fbuf_b, idxv, red, colbuf,
        shared, sem_a, sem_b):
    cid = lax.axis_index("c")
    sid = lax.axis_index("s")
    wid = cid * NS + sid
    base = wid * PW
    iota = lax.iota(I32, L)
    bufs = (fbuf_a, fbuf_b)
    sems = (sem_a, sem_b)

    def issue(s):
        buf, sem = bufs[s % 2], sems[s % 2]
        return [
            pltpu.async_copy(flow_hbm.at[s, c, pl.ds(base, PW)],
                             buf.at[pl.ds(c * PW, PW)], sem)
            for c in range(3)
        ]

    pend = issue(0)
    pltpu.sync_copy(imap_hbm.at[pl.ds(base, PW)], idxv)

    def zero_body(j, carry):
        acc[pl.ds(j * L, L)] = jnp.zeros((L,), F32)
        return carry

    lax.fori_loop(0, ACC_W // L, zero_body, 0)

    ones = jnp.ones((L,), F32)
    for s in range(S):
        for cp in pend:
            cp.wait()
        if s + 1 < S:
            pend = issue(s + 1)
        fbuf = bufs[s % 2]

        def body(g, carry, s=s, fbuf=fbuf):
            idx = idxv[pl.ds(g * L, L)]
            if s == 0:
                plsc.addupdate_scatter(acc, [idx + (R * K)], ones)
            for c in range(3):
                f = fbuf[pl.ds(c * PW + g * L, L)]
                plsc.addupdate_scatter(acc, [idx + ((s * 3 + c) * K)], f)
            return carry

        lax.fori_loop(0, NG, body, 0)

    pltpu.sync_copy(acc, shared.at[pl.ds(sid * ACC_W, ACC_W)])
    plsc.subcore_barrier()
    col0 = sid * COLS
    for l in range(NS):
        pltpu.sync_copy(shared.at[pl.ds(l * ACC_W + col0, COLS)],
                        red.at[pl.ds(l * COLS, COLS)])

    def red_body(t, carry):
        v = red[pl.ds(t * L, L)]
        for l in range(1, NS):
            v = v + red[pl.ds(l * COLS + t * L, L)]
        colbuf[pl.ds(t * L, L)] = v
        return carry

    lax.fori_loop(0, COLS // L, red_body, 0)
    pltpu.sync_copy(colbuf, out_hbm.at[pl.ds(cid * ACC_W + col0, COLS)])


@functools.partial(
    pl.kernel,
    out_type=jax.ShapeDtypeStruct((NC * K,), F32),
    mesh=_mesh,
    scratch_types=[
        pltpu.VMEM((NC * ACC_W,), F32),
        pltpu.VMEM((R * K,), F32),
        pltpu.VMEM((K,), F32),
        pltpu.VMEM((3 * PW,), F32),
        pltpu.VMEM((3 * PW,), F32),
        pltpu.VMEM((PW,), I32),
        pltpu.VMEM((K,), F32),
        pltpu.VMEM((NS * L,), F32),
        pltpu.VMEM_SHARED((NS * K,), F32),
        pltpu.SemaphoreType.DMA,
        pltpu.SemaphoreType.DMA,
    ],
    compiler_params=pltpu.CompilerParams(
        needs_layout_passes=False, use_tc_tiling_on_sc=False),
)
def _k2(flow_hbm, imap_hbm, part_hbm, out_hbm,
        pbuf, means, invc, fbuf_a, fbuf_b, idxv, nacc, red, sharedn,
        sem_a, sem_b):
    cid = lax.axis_index("c")
    sid = lax.axis_index("s")
    wid = cid * NS + sid
    base = wid * PW
    iota = lax.iota(I32, L)
    bufs = (fbuf_a, fbuf_b)
    sems = (sem_a, sem_b)

    def issue(s):
        buf, sem = bufs[s % 2], sems[s % 2]
        return [
            pltpu.async_copy(flow_hbm.at[s, c, pl.ds(base, PW)],
                             buf.at[pl.ds(c * PW, PW)], sem)
            for c in range(3)
        ]

    pend = issue(0)
    pltpu.sync_copy(imap_hbm.at[pl.ds(base, PW)], idxv)
    pltpu.sync_copy(part_hbm, pbuf)
    for q in range(K // L):
        cnt = (pbuf[pl.ds(R * K + q * L, L)]
               + pbuf[pl.ds(ACC_W + R * K + q * L, L)])
        invc[pl.ds(q * L, L)] = 1.0 / jnp.maximum(cnt, 1.0)

    def mean_body(j, carry):
        t = pbuf[pl.ds(j * L, L)] + pbuf[pl.ds(ACC_W + j * L, L)]
        q = lax.rem(j, K // L)
        means[pl.ds(j * L, L)] = t * invc[pl.ds(q * L, L)]
        return carry

    lax.fori_loop(0, R * K // L, mean_body, 0)

    for q in range(K // L):
        nacc[pl.ds(q * L, L)] = jnp.zeros((L,), F32)

    for s in range(S):
        for cp in pend:
            cp.wait()
        if s + 1 < S:
            pend = issue(s + 1)
        fbuf = bufs[s % 2]

        def body(g, carry, s=s, fbuf=fbuf):
            idx = idxv[pl.ds(g * L, L)]
            sq = jnp.zeros((L,), F32)
            for c in range(3):
                f = fbuf[pl.ds(c * PW + g * L, L)]
                m = plsc.load_gather(means, [idx + ((s * 3 + c) * K)])
                d = m - f
                sq = sq + d * d
            xc = jnp.maximum(sq, 1e-20)
            nrm = sq * _rsqrt_nr(xc)
            plsc.addupdate_scatter(nacc, [idx], nrm)
            return carry

        lax.fori_loop(0, NG, body, 0)

    pltpu.sync_copy(nacc, sharedn.at[pl.ds(sid * K, K)])
    plsc.subcore_barrier()

    @pl.when(sid < K // L)
    def _():
        col0 = sid * L
        for l in range(NS):
            pltpu.sync_copy(sharedn.at[pl.ds(l * K + col0, L)],
                            red.at[pl.ds(l * L, L)])
        v = red[pl.ds(0, L)]
        for l in range(1, NS):
            v = v + red[pl.ds(l * L, L)]
        nacc[pl.ds(0, L)] = v
        pltpu.sync_copy(nacc.at[pl.ds(0, L)],
                        out_hbm.at[pl.ds(cid * K + col0, L)])


def _final_body(part_ref, n_ref, out_ref):
    part = part_ref[...]
    nsum = n_ref[...]
    counts = part[0:1, R * K:R * K + K] + part[1:2, R * K:R * K + K]
    ns = nsum[0:1, :] + nsum[1:2, :]
    safe = jnp.maximum(counts, 1.0)
    per = ns / (safe * float(S))
    present = counts > 0.5
    zero = jnp.zeros_like(per)
    tot = jnp.sum(jnp.where(present, per, zero))
    npres = jnp.sum(jnp.where(present, jnp.ones_like(per), zero))
    out_ref[...] = jnp.full((1, 1), tot / (npres - 1.0), F32)


@jax.jit
def kernel(pred_flow, instance_map):
    flow_t = jnp.transpose(pred_flow, (0, 2, 1))
    part = _k1(flow_t, instance_map)
    nsum = _k2(flow_t, instance_map, part)
    out = pl.pallas_call(
        _final_body,
        out_shape=jax.ShapeDtypeStruct((1, 1), F32),
    )(jnp.reshape(part, (NC, ACC_W)), jnp.reshape(nsum, (NC, K)))
    return out[0, 0]


# speedup vs baseline: 48.0821x; 1.6408x over previous
import functools

import jax
import jax.numpy as jnp
from jax import lax
from jax.experimental import pallas as pl
from jax.experimental.pallas import tpu as pltpu
from jax.experimental.pallas import tpu_sc as plsc

S = 16
N = 65536
K = 64
NC = 2
NS = 16
L = 16
NW = NC * NS
PW = N // NW
NG = PW // L
R = S * 3
ACC_W = 3328
COLS = ACC_W // NS
F32 = jnp.float32
I32 = jnp.int32

_mesh = plsc.VectorSubcoreMesh(core_axis_name="c", subcore_axis_name="s")


def _rsqrt_nr(x):
    i = lax.bitcast_convert_type(x, I32)
    i = 0x5F3759DF - lax.shift_right_logical(i, 1)
    y = lax.bitcast_convert_type(i, F32)
    for _ in range(3):
        y = y * (1.5 - 0.5 * x * y * y)
    return y


@functools.partial(
    pl.kernel,
    out_type=jax.ShapeDtypeStruct((NC * ACC_W,), F32),
    mesh=_mesh,
    scratch_types=[
        pltpu.VMEM((ACC_W,), F32),
        pltpu.VMEM((3 * PW,), F32),
        pltpu.VMEM((3 * PW,), F32),
        pltpu.VMEM((PW,), I32),
        pltpu.VMEM((NS * COLS,), F32),
        pltpu.VMEM((COLS,), F32),
        pltpu.VMEM_SHARED((NS * ACC_W,), F32),
        pltpu.SemaphoreType.DMA,
        pltpu.SemaphoreType.DMA,
    ],
    compiler_params=pltpu.CompilerParams(
        needs_layout_passes=False, use_tc_tiling_on_sc=False),
)
def _k1(flow_hbm, imap_hbm, out_hbm, acc, fbuf_a, fbuf_b, idxv, red, colbuf,
        shared, sem_a, sem_b):
    cid = lax.axis_index("c")
    sid = lax.axis_index("s")
    wid = cid * NS + sid
    base = wid * PW
    iota = lax.iota(I32, L)
    bufs = (fbuf_a, fbuf_b)
    sems = (sem_a, sem_b)

    def issue(s):
        buf, sem = bufs[s % 2], sems[s % 2]
        return [
            pltpu.async_copy(flow_hbm.at[s, c, pl.ds(base, PW)],
                             buf.at[pl.ds(c * PW, PW)], sem)
            for c in range(3)
        ]

    pend = issue(0)
    pltpu.sync_copy(imap_hbm.at[pl.ds(base, PW)], idxv)

    @plsc.parallel_loop(0, ACC_W // L, unroll=4)
    def _zero(j):
        acc[pl.ds(j * L, L)] = jnp.zeros((L,), F32)

    ones = jnp.ones((L,), F32)
    for s in range(S):
        for cp in pend:
            cp.wait()
        if s + 1 < S:
            pend = issue(s + 1)
        fbuf = bufs[s % 2]

        @plsc.parallel_loop(0, NG, unroll=4)
        def _body(g, s=s, fbuf=fbuf):
            idx = idxv[pl.ds(g * L, L)]
            if s == 0:
                plsc.addupdate_scatter(acc, [idx + (R * K)], ones)
            for c in range(3):
                f = fbuf[pl.ds(c * PW + g * L, L)]
                plsc.addupdate_scatter(acc, [idx + ((s * 3 + c) * K)], f)

    pltpu.sync_copy(acc, shared.at[pl.ds(sid * ACC_W, ACC_W)])
    plsc.subcore_barrier()
    col0 = sid * COLS
    for l in range(NS):
        pltpu.sync_copy(shared.at[pl.ds(l * ACC_W + col0, COLS)],
                        red.at[pl.ds(l * COLS, COLS)])

    @plsc.parallel_loop(0, COLS // L, unroll=2)
    def _red(t):
        v = red[pl.ds(t * L, L)]
        for l in range(1, NS):
            v = v + red[pl.ds(l * COLS + t * L, L)]
        colbuf[pl.ds(t * L, L)] = v
    pltpu.sync_copy(colbuf, out_hbm.at[pl.ds(cid * ACC_W + col0, COLS)])


@functools.partial(
    pl.kernel,
    out_type=jax.ShapeDtypeStruct((NC * K,), F32),
    mesh=_mesh,
    scratch_types=[
        pltpu.VMEM((NC * ACC_W,), F32),
        pltpu.VMEM((R * K,), F32),
        pltpu.VMEM((K,), F32),
        pltpu.VMEM((3 * PW,), F32),
        pltpu.VMEM((3 * PW,), F32),
        pltpu.VMEM((PW,), I32),
        pltpu.VMEM((K,), F32),
        pltpu.VMEM((NS * L,), F32),
        pltpu.VMEM_SHARED((NS * K,), F32),
        pltpu.SemaphoreType.DMA,
        pltpu.SemaphoreType.DMA,
    ],
    compiler_params=pltpu.CompilerParams(
        needs_layout_passes=False, use_tc_tiling_on_sc=False),
)
def _k2(flow_hbm, imap_hbm, part_hbm, out_hbm,
        pbuf, means, invc, fbuf_a, fbuf_b, idxv, nacc, red, sharedn,
        sem_a, sem_b):
    cid = lax.axis_index("c")
    sid = lax.axis_index("s")
    wid = cid * NS + sid
    base = wid * PW
    iota = lax.iota(I32, L)
    bufs = (fbuf_a, fbuf_b)
    sems = (sem_a, sem_b)

    def issue(s):
        buf, sem = bufs[s % 2], sems[s % 2]
        return [
            pltpu.async_copy(flow_hbm.at[s, c, pl.ds(base, PW)],
                             buf.at[pl.ds(c * PW, PW)], sem)
            for c in range(3)
        ]

    pend = issue(0)
    pltpu.sync_copy(imap_hbm.at[pl.ds(base, PW)], idxv)
    pltpu.sync_copy(part_hbm, pbuf)
    for q in range(K // L):
        cnt = (pbuf[pl.ds(R * K + q * L, L)]
               + pbuf[pl.ds(ACC_W + R * K + q * L, L)])
        invc[pl.ds(q * L, L)] = 1.0 / jnp.maximum(cnt, 1.0)

    @plsc.parallel_loop(0, R * K // L, unroll=2)
    def _mean(j):
        t = pbuf[pl.ds(j * L, L)] + pbuf[pl.ds(ACC_W + j * L, L)]
        q = lax.rem(j, K // L)
        means[pl.ds(j * L, L)] = t * invc[pl.ds(q * L, L)]

    for q in range(K // L):
        nacc[pl.ds(q * L, L)] = jnp.zeros((L,), F32)

    for s in range(S):
        for cp in pend:
            cp.wait()
        if s + 1 < S:
            pend = issue(s + 1)
        fbuf = bufs[s % 2]

        @plsc.parallel_loop(0, NG, unroll=2)
        def _body(g, s=s, fbuf=fbuf):
            idx = idxv[pl.ds(g * L, L)]
            sq = jnp.zeros((L,), F32)
            for c in range(3):
                f = fbuf[pl.ds(c * PW + g * L, L)]
                m = plsc.load_gather(means, [idx + ((s * 3 + c) * K)])
                d = m - f
                sq = sq + d * d
            xc = jnp.maximum(sq, 1e-20)
            nrm = sq * _rsqrt_nr(xc)
            plsc.addupdate_scatter(nacc, [idx], nrm)

    pltpu.sync_copy(nacc, sharedn.at[pl.ds(sid * K, K)])
    plsc.subcore_barrier()

    @pl.when(sid < K // L)
    def _():
        col0 = sid * L
        for l in range(NS):
            pltpu.sync_copy(sharedn.at[pl.ds(l * K + col0, L)],
                            red.at[pl.ds(l * L, L)])
        v = red[pl.ds(0, L)]
        for l in range(1, NS):
            v = v + red[pl.ds(l * L, L)]
        nacc[pl.ds(0, L)] = v
        pltpu.sync_copy(nacc.at[pl.ds(0, L)],
                        out_hbm.at[pl.ds(cid * K + col0, L)])


def _final_body(part_ref, n_ref, out_ref):
    part = part_ref[...]
    nsum = n_ref[...]
    counts = part[0:1, R * K:R * K + K] + part[1:2, R * K:R * K + K]
    ns = nsum[0:1, :] + nsum[1:2, :]
    safe = jnp.maximum(counts, 1.0)
    per = ns / (safe * float(S))
    present = counts > 0.5
    zero = jnp.zeros_like(per)
    tot = jnp.sum(jnp.where(present, per, zero))
    npres = jnp.sum(jnp.where(present, jnp.ones_like(per), zero))
    out_ref[...] = jnp.full((1, 1), tot / (npres - 1.0), F32)


@jax.jit
def kernel(pred_flow, instance_map):
    flow_t = jnp.transpose(pred_flow, (0, 2, 1))
    part = _k1(flow_t, instance_map)
    nsum = _k2(flow_t, instance_map, part)
    out = pl.pallas_call(
        _final_body,
        out_shape=jax.ShapeDtypeStruct((1, 1), F32),
    )(jnp.reshape(part, (NC, ACC_W)), jnp.reshape(nsum, (NC, K)))
    return out[0, 0]


# speedup vs baseline: 78.3932x; 1.6304x over previous
import functools

import jax
import jax.numpy as jnp
from jax import lax
from jax.experimental import pallas as pl
from jax.experimental.pallas import tpu as pltpu
from jax.experimental.pallas import tpu_sc as plsc

S = 16
N = 65536
K = 64
NC = 2
NS = 16
L = 16
NW = NC * NS
PW = N // NW
NG = PW // L
R = S * 3
ACC_W = 3328
COLS = ACC_W // NS
F32 = jnp.float32
I32 = jnp.int32

_mesh = plsc.VectorSubcoreMesh(core_axis_name="c", subcore_axis_name="s")


def _rsqrt_nr(x):
    i = lax.bitcast_convert_type(x, I32)
    i = 0x5F3759DF - lax.shift_right_logical(i, 1)
    y = lax.bitcast_convert_type(i, F32)
    for _ in range(3):
        y = y * (1.5 - 0.5 * x * y * y)
    return y


@functools.partial(
    pl.kernel,
    out_type=jax.ShapeDtypeStruct((NC * ACC_W,), F32),
    mesh=_mesh,
    scratch_types=[
        pltpu.VMEM((ACC_W,), F32),
        pltpu.VMEM((R, 1, 128), F32),
        pltpu.VMEM((R, 1, 128), F32),
        pltpu.VMEM((PW,), I32),
        pltpu.VMEM((NS * COLS,), F32),
        pltpu.VMEM((COLS,), F32),
        pltpu.VMEM_SHARED((NS * ACC_W,), F32),
        pltpu.SemaphoreType.DMA,
        pltpu.SemaphoreType.DMA,
    ],
    compiler_params=pltpu.CompilerParams(
        needs_layout_passes=False, use_tc_tiling_on_sc=False),
)
def _k1(flow_hbm, imap_hbm, out_hbm, acc, fbuf_a, fbuf_b, idxv, red, colbuf,
        shared, sem_a, sem_b):
    cid = lax.axis_index("c")
    sid = lax.axis_index("s")
    wid = cid * NS + sid
    base = wid * PW
    tbase = wid * (PW // 128)
    iota = lax.iota(I32, L)
    bufs = (fbuf_a, fbuf_b)
    sems = (sem_a, sem_b)

    def issue(s):
        buf, sem = bufs[s % 2], sems[s % 2]
        a, b = s // 8, s % 8
        return [
            pltpu.async_copy(
                flow_hbm.at[c, a, pl.ds(tbase, PW // 128), pl.ds(b, 1), :],
                buf.at[pl.ds(c * (PW // 128), PW // 128)], sem)
            for c in range(3)
        ]

    pend = issue(0)
    pltpu.sync_copy(imap_hbm.at[pl.ds(base, PW)], idxv)

    @plsc.parallel_loop(0, ACC_W // L, unroll=4)
    def _zero(j):
        acc[pl.ds(j * L, L)] = jnp.zeros((L,), F32)

    ones = jnp.ones((L,), F32)
    for s in range(S):
        for cp in pend:
            cp.wait()
        if s + 1 < S:
            pend = issue(s + 1)
        fbuf = bufs[s % 2]

        @plsc.parallel_loop(0, NG, unroll=4)
        def _body(g, s=s, fbuf=fbuf):
            idx = idxv[pl.ds(g * L, L)]
            row = g >> 3
            col = (g & 7) * L
            if s == 0:
                plsc.addupdate_scatter(acc, [idx + (R * K)], ones)
            for c in range(3):
                f = fbuf[c * (PW // 128) + row, 0, pl.ds(col, L)]
                plsc.addupdate_scatter(acc, [idx + ((s * 3 + c) * K)], f)

    pltpu.sync_copy(acc, shared.at[pl.ds(sid * ACC_W, ACC_W)])
    plsc.subcore_barrier()
    col0 = sid * COLS
    for l in range(NS):
        pltpu.sync_copy(shared.at[pl.ds(l * ACC_W + col0, COLS)],
                        red.at[pl.ds(l * COLS, COLS)])

    @plsc.parallel_loop(0, COLS // L, unroll=2)
    def _red(t):
        v = red[pl.ds(t * L, L)]
        for l in range(1, NS):
            v = v + red[pl.ds(l * COLS + t * L, L)]
        colbuf[pl.ds(t * L, L)] = v
    pltpu.sync_copy(colbuf, out_hbm.at[pl.ds(cid * ACC_W + col0, COLS)])


@functools.partial(
    pl.kernel,
    out_type=jax.ShapeDtypeStruct((NC * K,), F32),
    mesh=_mesh,
    scratch_types=[
        pltpu.VMEM((NC * ACC_W,), F32),
        pltpu.VMEM((R * K,), F32),
        pltpu.VMEM((K,), F32),
        pltpu.VMEM((R, 1, 128), F32),
        pltpu.VMEM((R, 1, 128), F32),
        pltpu.VMEM((PW,), I32),
        pltpu.VMEM((K,), F32),
        pltpu.VMEM((NS * L,), F32),
        pltpu.VMEM_SHARED((NS * K,), F32),
        pltpu.SemaphoreType.DMA,
        pltpu.SemaphoreType.DMA,
    ],
    compiler_params=pltpu.CompilerParams(
        needs_layout_passes=False, use_tc_tiling_on_sc=False),
)
def _k2(flow_hbm, imap_hbm, part_hbm, out_hbm,
        pbuf, means, invc, fbuf_a, fbuf_b, idxv, nacc, red, sharedn,
        sem_a, sem_b):
    cid = lax.axis_index("c")
    sid = lax.axis_index("s")
    wid = cid * NS + sid
    base = wid * PW
    tbase = wid * (PW // 128)
    iota = lax.iota(I32, L)
    bufs = (fbuf_a, fbuf_b)
    sems = (sem_a, sem_b)

    def issue(s):
        buf, sem = bufs[s % 2], sems[s % 2]
        a, b = s // 8, s % 8
        return [
            pltpu.async_copy(
                flow_hbm.at[c, a, pl.ds(tbase, PW // 128), pl.ds(b, 1), :],
                buf.at[pl.ds(c * (PW // 128), PW // 128)], sem)
            for c in range(3)
        ]

    pend = issue(0)
    pltpu.sync_copy(imap_hbm.at[pl.ds(base, PW)], idxv)
    pltpu.sync_copy(part_hbm, pbuf)
    for q in range(K // L):
        cnt = (pbuf[pl.ds(R * K + q * L, L)]
               + pbuf[pl.ds(ACC_W + R * K + q * L, L)])
        invc[pl.ds(q * L, L)] = 1.0 / jnp.maximum(cnt, 1.0)

    @plsc.parallel_loop(0, R * K // L, unroll=2)
    def _mean(j):
        t = pbuf[pl.ds(j * L, L)] + pbuf[pl.ds(ACC_W + j * L, L)]
        q = lax.rem(j, K // L)
        means[pl.ds(j * L, L)] = t * invc[pl.ds(q * L, L)]

    for q in range(K // L):
        nacc[pl.ds(q * L, L)] = jnp.zeros((L,), F32)

    for s in range(S):
        for cp in pend:
            cp.wait()
        if s + 1 < S:
            pend = issue(s + 1)
        fbuf = bufs[s % 2]

        @plsc.parallel_loop(0, NG, unroll=2)
        def _body(g, s=s, fbuf=fbuf):
            idx = idxv[pl.ds(g * L, L)]
            row = g >> 3
            col = (g & 7) * L
            sq = jnp.zeros((L,), F32)
            for c in range(3):
                f = fbuf[c * (PW // 128) + row, 0, pl.ds(col, L)]
                m = plsc.load_gather(means, [idx + ((s * 3 + c) * K)])
                d = m - f
                sq = sq + d * d
            xc = jnp.maximum(sq, 1e-20)
            nrm = sq * _rsqrt_nr(xc)
            plsc.addupdate_scatter(nacc, [idx], nrm)

    pltpu.sync_copy(nacc, sharedn.at[pl.ds(sid * K, K)])
    plsc.subcore_barrier()

    @pl.when(sid < K // L)
    def _():
        col0 = sid * L
        for l in range(NS):
            pltpu.sync_copy(sharedn.at[pl.ds(l * K + col0, L)],
                            red.at[pl.ds(l * L, L)])
        v = red[pl.ds(0, L)]
        for l in range(1, NS):
            v = v + red[pl.ds(l * L, L)]
        nacc[pl.ds(0, L)] = v
        pltpu.sync_copy(nacc.at[pl.ds(0, L)],
                        out_hbm.at[pl.ds(cid * K + col0, L)])


def _final_body(part_ref, n_ref, out_ref):
    part = part_ref[...]
    nsum = n_ref[...]
    counts = part[0:1, R * K:R * K + K] + part[1:2, R * K:R * K + K]
    ns = nsum[0:1, :] + nsum[1:2, :]
    safe = jnp.maximum(counts, 1.0)
    per = ns / (safe * float(S))
    present = counts > 0.5
    zero = jnp.zeros_like(per)
    tot = jnp.sum(jnp.where(present, per, zero))
    npres = jnp.sum(jnp.where(present, jnp.ones_like(per), zero))
    out_ref[...] = jnp.full((1, 1), tot / (npres - 1.0), F32)


@jax.jit
def kernel(pred_flow, instance_map):
    flow5 = jnp.transpose(
        jnp.reshape(pred_flow, (2, 8, 512, 128, 3)), (4, 0, 2, 1, 3))
    part = _k1(flow5, instance_map)
    nsum = _k2(flow5, instance_map, part)
    out = pl.pallas_call(
        _final_body,
        out_shape=jax.ShapeDtypeStruct((1, 1), F32),
    )(jnp.reshape(part, (NC, ACC_W)), jnp.reshape(nsum, (NC, K)))
    return out[0, 0]


# speedup vs baseline: 80.5331x; 1.0273x over previous
import functools

import jax
import jax.numpy as jnp
from jax import lax
from jax.experimental import pallas as pl
from jax.experimental.pallas import tpu as pltpu
from jax.experimental.pallas import tpu_sc as plsc

S = 16
N = 65536
K = 64
NC = 2
NS = 16
L = 16
NW = NC * NS
PW = N // NW
NG = PW // L
R = S * 3
ACC_W = 3328
COLS = ACC_W // NS
F32 = jnp.float32
I32 = jnp.int32

_mesh = plsc.VectorSubcoreMesh(core_axis_name="c", subcore_axis_name="s")


def _rsqrt_nr(x):
    i = lax.bitcast_convert_type(x, I32)
    i = 0x5F3759DF - lax.shift_right_logical(i, 1)
    y = lax.bitcast_convert_type(i, F32)
    xh = 0.5 * x
    for _ in range(2):
        y = y * (1.5 - xh * y * y)
    return y


@functools.partial(
    pl.kernel,
    out_type=jax.ShapeDtypeStruct((NC * ACC_W,), F32),
    mesh=_mesh,
    scratch_types=[
        pltpu.VMEM((ACC_W,), F32),
        pltpu.VMEM((R, 1, 128), F32),
        pltpu.VMEM((R, 1, 128), F32),
        pltpu.VMEM((PW,), I32),
        pltpu.VMEM((NS * COLS,), F32),
        pltpu.VMEM((COLS,), F32),
        pltpu.VMEM_SHARED((NS * ACC_W,), F32),
        pltpu.SemaphoreType.DMA,
        pltpu.SemaphoreType.DMA,
    ],
    compiler_params=pltpu.CompilerParams(
        needs_layout_passes=False, use_tc_tiling_on_sc=False),
)
def _k1(flow_hbm, imap_hbm, out_hbm, acc, fbuf_a, fbuf_b, idxv, red, colbuf,
        shared, sem_a, sem_b):
    cid = lax.axis_index("c")
    sid = lax.axis_index("s")
    wid = cid * NS + sid
    base = wid * PW
    tbase = wid * (PW // 128)
    iota = lax.iota(I32, L)
    bufs = (fbuf_a, fbuf_b)
    sems = (sem_a, sem_b)

    def issue(s):
        buf, sem = bufs[s % 2], sems[s % 2]
        a, b = s // 8, s % 8
        return [
            pltpu.async_copy(
                flow_hbm.at[c, a, pl.ds(tbase, PW // 128), pl.ds(b, 1), :],
                buf.at[pl.ds(c * (PW // 128), PW // 128)], sem)
            for c in range(3)
        ]

    pend = issue(0)
    pltpu.sync_copy(imap_hbm.at[pl.ds(base, PW)], idxv)

    @plsc.parallel_loop(0, ACC_W // L, unroll=4)
    def _zero(j):
        acc[pl.ds(j * L, L)] = jnp.zeros((L,), F32)

    ones = jnp.ones((L,), F32)
    for s in range(S):
        for cp in pend:
            cp.wait()
        if s + 1 < S:
            pend = issue(s + 1)
        fbuf = bufs[s % 2]

        @plsc.parallel_loop(0, NG, unroll=4)
        def _body(g, s=s, fbuf=fbuf):
            idx = idxv[pl.ds(g * L, L)]
            row = g >> 3
            col = (g & 7) * L
            if s == 0:
                plsc.addupdate_scatter(acc, [idx + (R * K)], ones)
            for c in range(3):
                f = fbuf[c * (PW // 128) + row, 0, pl.ds(col, L)]
                plsc.addupdate_scatter(acc, [idx + ((s * 3 + c) * K)], f)

    pltpu.sync_copy(acc, shared.at[pl.ds(sid * ACC_W, ACC_W)])
    plsc.subcore_barrier()
    col0 = sid * COLS
    for l in range(NS):
        pltpu.sync_copy(shared.at[pl.ds(l * ACC_W + col0, COLS)],
                        red.at[pl.ds(l * COLS, COLS)])

    @plsc.parallel_loop(0, COLS // L, unroll=2)
    def _red(t):
        v = red[pl.ds(t * L, L)]
        for l in range(1, NS):
            v = v + red[pl.ds(l * COLS + t * L, L)]
        colbuf[pl.ds(t * L, L)] = v
    pltpu.sync_copy(colbuf, out_hbm.at[pl.ds(cid * ACC_W + col0, COLS)])


@functools.partial(
    pl.kernel,
    out_type=jax.ShapeDtypeStruct((NC * K,), F32),
    mesh=_mesh,
    scratch_types=[
        pltpu.VMEM((NC * ACC_W,), F32),
        pltpu.VMEM((R * K,), F32),
        pltpu.VMEM((K,), F32),
        pltpu.VMEM((R * (PW // 256), 1, 128), F32),
        pltpu.VMEM((R * (PW // 256), 1, 128), F32),
        pltpu.VMEM((PW,), I32),
        pltpu.VMEM((K,), F32),
        pltpu.VMEM((NS * L,), F32),
        pltpu.VMEM_SHARED((NS * K,), F32),
        pltpu.SemaphoreType.DMA,
        pltpu.SemaphoreType.DMA,
    ],
    compiler_params=pltpu.CompilerParams(
        needs_layout_passes=False, use_tc_tiling_on_sc=False),
)
def _k2(flow_hbm, imap_hbm, part_hbm, out_hbm,
        pbuf, means, invc, fbuf_a, fbuf_b, idxv, nacc, red, sharedn,
        sem_a, sem_b):
    cid = lax.axis_index("c")
    sid = lax.axis_index("s")
    wid = cid * NS + sid
    base = wid * PW
    tbase = wid * (PW // 128)
    iota = lax.iota(I32, L)
    bufs = (fbuf_a, fbuf_b)
    sems = (sem_a, sem_b)

    HT = PW // 256

    def issue(h):
        buf, sem = bufs[h % 2], sems[h % 2]
        cps = []
        for s in range(S):
            a, b = s // 8, s % 8
            for c in range(3):
                cps.append(pltpu.async_copy(
                    flow_hbm.at[c, a, pl.ds(tbase + h * HT, HT),
                                pl.ds(b, 1), :],
                    buf.at[pl.ds((s * 3 + c) * HT, HT)], sem))
        return cps

    pend = issue(0)
    pltpu.sync_copy(imap_hbm.at[pl.ds(base, PW)], idxv)
    pltpu.sync_copy(part_hbm, pbuf)
    for q in range(K // L):
        cnt = (pbuf[pl.ds(R * K + q * L, L)]
               + pbuf[pl.ds(ACC_W + R * K + q * L, L)])
        invc[pl.ds(q * L, L)] = 1.0 / jnp.maximum(cnt, 1.0)

    @plsc.parallel_loop(0, R * K // L, unroll=2)
    def _mean(j):
        t = pbuf[pl.ds(j * L, L)] + pbuf[pl.ds(ACC_W + j * L, L)]
        q = lax.rem(j, K // L)
        means[pl.ds(j * L, L)] = t * invc[pl.ds(q * L, L)]

    for q in range(K // L):
        nacc[pl.ds(q * L, L)] = jnp.zeros((L,), F32)

    for h in range(2):
        for cp in pend:
            cp.wait()
        if h == 0:
            pend = issue(1)
        fbuf = bufs[h % 2]

        @plsc.parallel_loop(0, NG // 2, unroll=2)
        def _body(g, h=h, fbuf=fbuf):
            idx = idxv[pl.ds((h * (NG // 2) + g) * L, L)]
            row = g >> 3
            col = (g & 7) * L
            acc_n = jnp.zeros((L,), F32)
            for s in range(S):
                sq = jnp.zeros((L,), F32)
                for c in range(3):
                    f = fbuf[(s * 3 + c) * HT + row, 0, pl.ds(col, L)]
                    m = plsc.load_gather(means, [idx + ((s * 3 + c) * K)])
                    d = m - f
                    sq = sq + d * d
                xc = jnp.maximum(sq, 1e-20)
                acc_n = acc_n + sq * _rsqrt_nr(xc)
            plsc.addupdate_scatter(nacc, [idx], acc_n)

    pltpu.sync_copy(nacc, sharedn.at[pl.ds(sid * K, K)])
    plsc.subcore_barrier()

    @pl.when(sid < K // L)
    def _():
        col0 = sid * L
        for l in range(NS):
            pltpu.sync_copy(sharedn.at[pl.ds(l * K + col0, L)],
                            red.at[pl.ds(l * L, L)])
        v = red[pl.ds(0, L)]
        for l in range(1, NS):
            v = v + red[pl.ds(l * L, L)]
        nacc[pl.ds(0, L)] = v
        pltpu.sync_copy(nacc.at[pl.ds(0, L)],
                        out_hbm.at[pl.ds(cid * K + col0, L)])


def _final_body(part_ref, n_ref, out_ref):
    part = part_ref[...]
    nsum = n_ref[...]
    counts = part[0:1, R * K:R * K + K] + part[1:2, R * K:R * K + K]
    ns = nsum[0:1, :] + nsum[1:2, :]
    safe = jnp.maximum(counts, 1.0)
    per = ns / (safe * float(S))
    present = counts > 0.5
    zero = jnp.zeros_like(per)
    tot = jnp.sum(jnp.where(present, per, zero))
    npres = jnp.sum(jnp.where(present, jnp.ones_like(per), zero))
    out_ref[...] = jnp.full((1, 1), tot / (npres - 1.0), F32)


@jax.jit
def kernel(pred_flow, instance_map):
    flow5 = jnp.transpose(
        jnp.reshape(pred_flow, (2, 8, 512, 128, 3)), (4, 0, 2, 1, 3))
    part = _k1(flow5, instance_map)
    nsum = _k2(flow5, instance_map, part)
    out = pl.pallas_call(
        _final_body,
        out_shape=jax.ShapeDtypeStruct((1, 1), F32),
    )(jnp.reshape(part, (NC, ACC_W)), jnp.reshape(nsum, (NC, K)))
    return out[0, 0]
